# baseline (device time: 46184 ns/iter reference)
import os

import jax
import jax.numpy as jnp
from jax import lax
from jax.experimental import pallas as pl
from jax.experimental.pallas import tpu as pltpu

_KVAR = os.environ.get("KVAR", "full")

N_DEV = 8
B = 64
D = 512
H = 1024
NW = 2
WH = H // NW
HC = WH // N_DEV


def kernel(x, Win0, Wout0, Win1, Wout1, Win2, Wout2):
    def body(x_ref, win0_ref, wout0_ref, win1_ref, wout1_ref, win2_ref,
             wout2_ref, out_ref, partial_ref, recva_ref, h_ref, red_ref,
             senda_sems, recva_sems, sendb_sems, recvb_sems):
        my = lax.axis_index("i")
        slot_mask = lax.broadcasted_iota(jnp.int32, (N_DEV, 1, 1), 0) == my

        if _KVAR == "full":
            bar = pltpu.get_barrier_semaphore()
            for off in range(1, N_DEV):
                t = lax.rem(my + off, N_DEV)
                pl.semaphore_signal(bar, inc=1, device_id=(t,),
                                    device_id_type=pl.DeviceIdType.MESH)

        wins = [win0_ref, win1_ref, win2_ref]
        wouts = [wout0_ref, wout1_ref, wout2_ref]

        def start_scatter(w):
            rdmas = []
            for off in range(1, N_DEV):
                k = off - 1
                t = lax.rem(my + off, N_DEV)
                rdma = pltpu.make_async_remote_copy(
                    src_ref=partial_ref.at[w, t],
                    dst_ref=recva_ref.at[w, k],
                    send_sem=senda_sems.at[w * (N_DEV - 1) + k],
                    recv_sem=recva_sems.at[w * (N_DEV - 1) + k],
                    device_id=(t,),
                    device_id_type=pl.DeviceIdType.MESH,
                )
                rdma.start()
                rdmas.append(rdma)
            return rdmas

        def start_bcast(w):
            rdmas = []
            for off in range(1, N_DEV):
                k = off - 1
                t = lax.rem(my + off, N_DEV)
                rdma = pltpu.make_async_remote_copy(
                    src_ref=red_ref.at[w],
                    dst_ref=h_ref.at[w, my],
                    send_sem=sendb_sems.at[w * (N_DEV - 1) + k],
                    recv_sem=recvb_sems.at[w * (N_DEV - 1) + k],
                    device_id=(t,),
                    device_id_type=pl.DeviceIdType.MESH,
                )
                rdma.start()
                rdmas.append(rdma)
            return rdmas

        x_cur = x_ref[:, :]
        for l in range(3):
            win = wins[l][:, :]
            wout = wouts[l][:, :]

            p3 = [None] * NW
            rdmas_a = [[] for _ in range(NW)]
            rdmas_b = [[] for _ in range(NW)]

            for w in range(NW):
                pw = jnp.dot(x_cur, win[:, w * WH:(w + 1) * WH],
                             preferred_element_type=jnp.float32)
                p3[w] = jnp.swapaxes(pw.reshape(B, N_DEV, HC), 0, 1)
                partial_ref[w] = p3[w]
                if _KVAR == "nocomm" and l == 0:
                    recva_ref[w] = p3[w][: N_DEV - 1]
                    h_ref[w] = p3[w]
                if _KVAR == "full":
                    if l == 0 and w == 0:
                        pl.semaphore_wait(bar, N_DEV - 1)
                    rdmas_a[w] = start_scatter(w)

            hred = [None] * NW
            for w in range(NW):
                acc = jnp.sum(jnp.where(slot_mask, p3[w], 0.0), axis=0)
                for k in range(N_DEV - 1):
                    if rdmas_a[w]:
                        rdmas_a[w][k].wait_recv()
                    acc = acc + recva_ref[w, k]
                hred[w] = jnp.maximum(acc, 0.0)
                red_ref[w] = hred[w]
                if _KVAR == "full":
                    rdmas_b[w] = start_bcast(w)

            x_acc = None
            for w in range(NW):
                for r in rdmas_b[w]:
                    r.wait_recv()
                h3 = jnp.where(slot_mask, hred[w][None], h_ref[w])
                hw = jnp.swapaxes(h3, 0, 1).reshape(B, WH)
                part = jnp.dot(hw, wout[w * WH:(w + 1) * WH, :],
                               preferred_element_type=jnp.float32)
                x_acc = part if x_acc is None else x_acc + part
            x_cur = x_acc

            for w in range(NW):
                for r in rdmas_a[w]:
                    r.wait_send()
                for r in rdmas_b[w]:
                    r.wait_send()

        out_ref[:, :] = x_cur

    n_sems = NW * (N_DEV - 1)
    return pl.pallas_call(
        body,
        out_shape=jax.ShapeDtypeStruct((B, D), jnp.float32),
        in_specs=[pl.BlockSpec(memory_space=pltpu.VMEM)] * 7,
        out_specs=pl.BlockSpec(memory_space=pltpu.VMEM),
        scratch_shapes=[
            pltpu.VMEM((NW, N_DEV, B, HC), jnp.float32),
            pltpu.VMEM((NW, N_DEV - 1, B, HC), jnp.float32),
            pltpu.VMEM((NW, N_DEV, B, HC), jnp.float32),
            pltpu.VMEM((NW, B, HC), jnp.float32),
            pltpu.SemaphoreType.DMA((n_sems,)),
            pltpu.SemaphoreType.DMA((n_sems,)),
            pltpu.SemaphoreType.DMA((n_sems,)),
            pltpu.SemaphoreType.DMA((n_sems,)),
        ],
        compiler_params=(
            pltpu.CompilerParams(collective_id=0)
            if _KVAR == "full" else pltpu.CompilerParams()
        ),
    )(x, Win0, Wout0, Win1, Wout1, Win2, Wout2)


# device time: 11111 ns/iter; 4.1566x vs baseline; 4.1566x over previous
import os

import jax
import jax.numpy as jnp
from jax import lax
from jax.experimental import pallas as pl
from jax.experimental.pallas import tpu as pltpu

_KVAR = os.environ.get("KVAR", "full")

N_DEV = 8
B = 64
D = 512
H = 1024
HC = H // N_DEV


def kernel(x, Win0, Wout0, Win1, Wout1, Win2, Wout2):
    if _KVAR == "empty":
        def _copy_body(x_ref, *refs):
            refs[-1][:, :] = x_ref[:, :]

        return pl.pallas_call(
            _copy_body,
            out_shape=jax.ShapeDtypeStruct((B, D), jnp.float32),
            in_specs=[pl.BlockSpec(memory_space=pltpu.VMEM)] * 7,
            out_specs=pl.BlockSpec(memory_space=pltpu.VMEM),
        )(x, Win0, Wout0, Win1, Wout1, Win2, Wout2)

    def body(x_ref, win0_ref, wout0_ref, win1_ref, wout1_ref, win2_ref,
             wout2_ref, out_ref, partial_ref, recva_ref, h_ref, red_ref,
             senda_sems, recva_sems, sendb_sems, recvb_sems, local_sem):
        my = lax.axis_index("i")
        slot_mask = lax.broadcasted_iota(jnp.int32, (N_DEV, 1, 1), 0) == my

        if _KVAR == "full":
            bar = pltpu.get_barrier_semaphore()
            for off in range(1, N_DEV):
                t = lax.rem(my + off, N_DEV)
                pl.semaphore_signal(bar, inc=1, device_id=(t,),
                                    device_id_type=pl.DeviceIdType.MESH)

        wins = [win0_ref, win1_ref, win2_ref]
        wouts = [wout0_ref, wout1_ref, wout2_ref]

        x_cur = x_ref[:, :]
        for l in range(3):
            partial = jnp.dot(x_cur, wins[l][:, :],
                              preferred_element_type=jnp.float32)
            p3 = jnp.swapaxes(partial.reshape(B, N_DEV, HC), 0, 1)
            partial_ref[:, :, :] = p3
            if _KVAR == "nocomm" and l == 0:
                recva_ref[:, :, :] = p3[: N_DEV - 1]
                h_ref[:, :, :] = p3
            if _KVAR == "full" and l == 0:
                pl.semaphore_wait(bar, N_DEV - 1)

            rdmas_a = []
            if _KVAR == "full":
                for off in range(1, N_DEV):
                    k = off - 1
                    t = lax.rem(my + off, N_DEV)
                    rdma = pltpu.make_async_remote_copy(
                        src_ref=partial_ref.at[t],
                        dst_ref=recva_ref.at[k],
                        send_sem=senda_sems.at[k],
                        recv_sem=recva_sems.at[k],
                        device_id=(t,),
                        device_id_type=pl.DeviceIdType.MESH,
                    )
                    rdma.start()
                    rdmas_a.append(rdma)
            acc = jnp.sum(jnp.where(slot_mask, p3, 0.0), axis=0)
            for k in range(N_DEV - 1):
                if rdmas_a:
                    rdmas_a[k].wait_recv()
                acc = acc + recva_ref[k]
            hred = jnp.maximum(acc, 0.0)
            red_ref[:, :] = hred

            rdmas_b = []
            if _KVAR == "full":
                for off in range(1, N_DEV):
                    k = off - 1
                    t = lax.rem(my + off, N_DEV)
                    rdma = pltpu.make_async_remote_copy(
                        src_ref=red_ref,
                        dst_ref=h_ref.at[my],
                        send_sem=sendb_sems.at[k],
                        recv_sem=recvb_sems.at[k],
                        device_id=(t,),
                        device_id_type=pl.DeviceIdType.MESH,
                    )
                    rdma.start()
                    rdmas_b.append(rdma)
            for r in rdmas_b:
                r.wait_recv()

            h3 = jnp.where(slot_mask, hred[None, :, :], h_ref[:, :, :])
            h_full = jnp.swapaxes(h3, 0, 1).reshape(B, H)
            x_cur = jnp.dot(h_full, wouts[l][:, :],
                            preferred_element_type=jnp.float32)

            for r in rdmas_a:
                r.wait_send()
            for r in rdmas_b:
                r.wait_send()

        out_ref[:, :] = x_cur

    return pl.pallas_call(
        body,
        out_shape=jax.ShapeDtypeStruct((B, D), jnp.float32),
        in_specs=[pl.BlockSpec(memory_space=pltpu.VMEM)] * 7,
        out_specs=pl.BlockSpec(memory_space=pltpu.VMEM),
        scratch_shapes=[
            pltpu.VMEM((N_DEV, B, HC), jnp.float32),
            pltpu.VMEM((N_DEV - 1, B, HC), jnp.float32),
            pltpu.VMEM((N_DEV, B, HC), jnp.float32),
            pltpu.VMEM((B, HC), jnp.float32),
            pltpu.SemaphoreType.DMA((N_DEV - 1,)),
            pltpu.SemaphoreType.DMA((N_DEV - 1,)),
            pltpu.SemaphoreType.DMA((N_DEV - 1,)),
            pltpu.SemaphoreType.DMA((N_DEV - 1,)),
            pltpu.SemaphoreType.DMA,
        ],
        compiler_params=(
            pltpu.CompilerParams(collective_id=0)
            if _KVAR == "full" else pltpu.CompilerParams()
        ),
    )(x, Win0, Wout0, Win1, Wout1, Win2, Wout2)
